# baseline (speedup 1.0000x reference)
import functools

import jax
import jax.numpy as jnp
from jax import lax
from jax.experimental import pallas as pl
from jax.experimental.pallas import tpu as pltpu
from jax.experimental.pallas import tpu_sc as plsc

_D = 768
_MAX_REL = 125
_VOCAB = 2 * _MAX_REL + 1
_BLK = 1024
_S = 2048
_NW = 32
_JB = _S // (_NW // 2)
_DH = _D // 2
_CH = 32
_NCH = _JB // _CH
_NV = _DH // 16


def _sc_body(table_hbm, out_hbm, tbl_v, out_v):
    c = lax.axis_index("c")
    s = lax.axis_index("s")
    wid = s * 2 + c
    jb = wid // 2
    h = wid % 2
    j0 = jb * _JB
    h0 = h * _DH
    pltpu.sync_copy(table_hbm.at[:, pl.ds(h0, _DH)], tbl_v)

    coef0 = jnp.maximum(0, (_S - _MAX_REL) - j0).astype(jnp.float32)
    coef1 = jnp.maximum(0, j0 - (_MAX_REL - 1)).astype(jnp.float32)
    kmin = jnp.maximum(1, j0 - (_S - _MAX_REL - 1))
    kmax = jnp.minimum(_VOCAB - 2, j0 + _MAX_REL)

    accs = [coef0 * tbl_v[0, pl.ds(ci * 16, 16)]
            + coef1 * tbl_v[_VOCAB - 1, pl.ds(ci * 16, 16)]
            for ci in range(_NV)]

    def base_body(k, accs):
        return [a + tbl_v[k, pl.ds(ci * 16, 16)] for ci, a in enumerate(accs)]

    accs = lax.fori_loop(kmin, kmax + 1, base_body, accs)

    for ch in range(_NCH):
        def j_body(jj, accs):
            for ci in range(_NV):
                out_v[jj, pl.ds(ci * 16, 16)] = accs[ci]
            j = j0 + ch * _CH + jj
            kadd = jnp.minimum(_VOCAB - 1, j + _MAX_REL + 1)
            ksub = jnp.maximum(0, j - (_S - _MAX_REL - 1))
            return [a + tbl_v[kadd, pl.ds(ci * 16, 16)]
                    - tbl_v[ksub, pl.ds(ci * 16, 16)]
                    for ci, a in enumerate(accs)]

        accs = lax.fori_loop(0, _CH, j_body, accs)
        pltpu.sync_copy(out_v,
                        out_hbm.at[pl.ds(j0 + ch * _CH, _CH), pl.ds(h0, _DH)])


def _pooled_raw_sc(table):
    mesh = plsc.VectorSubcoreMesh(core_axis_name="c", subcore_axis_name="s")
    k = functools.partial(
        pl.kernel,
        mesh=mesh,
        out_type=jax.ShapeDtypeStruct((_S, _D), jnp.float32),
        scratch_types=[
            pltpu.VMEM((_VOCAB, _DH), jnp.float32),
            pltpu.VMEM((_CH, _DH), jnp.float32),
        ],
    )(_sc_body)
    return k(table)


def _tc_add_body(x_ref, pooled_ref, out_ref, *, S):
    out_ref[...] = x_ref[...] + pooled_ref[...][None, :, :] * (1.0 / S)


def kernel(x, table):
    B, S, d = x.shape
    pooled_raw = _pooled_raw_sc(table)
    grid = (S // _BLK,)
    body = functools.partial(_tc_add_body, S=S)
    return pl.pallas_call(
        body,
        grid=grid,
        in_specs=[
            pl.BlockSpec((B, _BLK, d), lambda s: (0, s, 0)),
            pl.BlockSpec((_BLK, d), lambda s: (s, 0)),
        ],
        out_specs=pl.BlockSpec((B, _BLK, d), lambda s: (0, s, 0)),
        out_shape=jax.ShapeDtypeStruct((B, S, d), x.dtype),
    )(x, pooled_raw)


# speedup vs baseline: 4.5574x; 4.5574x over previous
import functools

import jax
import jax.numpy as jnp
from jax.experimental import pallas as pl
from jax.experimental.pallas import tpu as pltpu

_D = 768
_MAX_REL = 125
_VOCAB = 2 * _MAX_REL + 1
_CH = 512
_NB = 3


def _body(x_hbm, table_ref, out_hbm, xbuf, obuf, insems, outsems, *, S, N):
    nch = N // _CH
    tbl = table_ref[...]

    def load(i):
        sl = i % _NB
        return pltpu.make_async_copy(
            x_hbm.at[pl.ds(i * _CH, _CH), :], xbuf.at[sl], insems.at[sl])

    def store(i):
        sl = i % _NB
        return pltpu.make_async_copy(
            obuf.at[sl], out_hbm.at[pl.ds(i * _CH, _CH), :], outsems.at[sl])

    for i in range(min(_NB, nch)):
        load(i).start()

    for i in range(nch):
        sl = i % _NB
        load(i).wait()
        rows = i * _CH + jax.lax.broadcasted_iota(jnp.int32, (_CH, _VOCAB), 0)
        jj = jax.lax.bitwise_and(rows, S - 1)
        kk = jax.lax.broadcasted_iota(jnp.int32, (_CH, _VOCAB), 1)
        interior = ((kk >= 1) & (kk <= _VOCAB - 2)
                    & (jj >= kk - _MAX_REL) & (jj <= kk - _MAX_REL + S - 1))
        cnt = jnp.where(kk == 0, jnp.maximum(0, (S - _MAX_REL) - jj), 0)
        cnt = cnt + jnp.where(kk == _VOCAB - 1,
                              jnp.maximum(0, jj - (_MAX_REL - 1)), 0)
        cnt = cnt + interior.astype(jnp.int32)
        c = cnt.astype(jnp.float32) * (1.0 / S)
        pooled = jax.lax.dot_general(
            c, tbl,
            dimension_numbers=(((1,), (0,)), ((), ())),
            preferred_element_type=jnp.float32,
        )
        if i >= _NB:
            store(i - _NB).wait()
        obuf[sl] = xbuf[sl] + pooled
        store(i).start()
        if i + _NB < nch:
            load(i + _NB).start()

    for i in range(max(0, nch - _NB), nch):
        store(i).wait()


def kernel(x, table):
    B, S, d = x.shape
    V = table.shape[0]
    N = B * S
    xf = x.reshape(N, d)
    body = functools.partial(_body, S=S, N=N)
    out = pl.pallas_call(
        body,
        in_specs=[
            pl.BlockSpec(memory_space=pl.ANY),
            pl.BlockSpec((V, d), lambda: (0, 0)),
        ],
        out_specs=pl.BlockSpec(memory_space=pl.ANY),
        out_shape=jax.ShapeDtypeStruct((N, d), x.dtype),
        scratch_shapes=[
            pltpu.VMEM((_NB, _CH, d), jnp.float32),
            pltpu.VMEM((_NB, _CH, d), jnp.float32),
            pltpu.SemaphoreType.DMA((_NB,)),
            pltpu.SemaphoreType.DMA((_NB,)),
        ],
    )(xf, table)
    return out.reshape(B, S, d)


# speedup vs baseline: 5.0495x; 1.1080x over previous
import functools

import jax
import jax.numpy as jnp
from jax.experimental import pallas as pl
from jax.experimental.pallas import tpu as pltpu

_D = 768
_MAX_REL = 125
_VOCAB = 2 * _MAX_REL + 1
_CH = 1024
_NB = 3


def _body(x_hbm, table_ref, out_hbm, xbuf, obuf, insems, outsems, *, S, N):
    nch = N // _CH
    tbl = table_ref[...]

    def load(i):
        sl = i % _NB
        return pltpu.make_async_copy(
            x_hbm.at[pl.ds(i * _CH, _CH), :], xbuf.at[sl], insems.at[sl])

    def store(i):
        sl = i % _NB
        return pltpu.make_async_copy(
            obuf.at[sl], out_hbm.at[pl.ds(i * _CH, _CH), :], outsems.at[sl])

    for i in range(min(_NB, nch)):
        load(i).start()

    for i in range(nch):
        sl = i % _NB
        load(i).wait()
        rows = i * _CH + jax.lax.broadcasted_iota(jnp.int32, (_CH, _VOCAB), 0)
        jj = jax.lax.bitwise_and(rows, S - 1)
        kk = jax.lax.broadcasted_iota(jnp.int32, (_CH, _VOCAB), 1)
        interior = ((kk >= 1) & (kk <= _VOCAB - 2)
                    & (jj >= kk - _MAX_REL) & (jj <= kk - _MAX_REL + S - 1))
        cnt = jnp.where(kk == 0, jnp.maximum(0, (S - _MAX_REL) - jj), 0)
        cnt = cnt + jnp.where(kk == _VOCAB - 1,
                              jnp.maximum(0, jj - (_MAX_REL - 1)), 0)
        cnt = cnt + interior.astype(jnp.int32)
        c = cnt.astype(jnp.float32) * (1.0 / S)
        pooled = jax.lax.dot_general(
            c, tbl,
            dimension_numbers=(((1,), (0,)), ((), ())),
            preferred_element_type=jnp.float32,
        )
        if i >= _NB:
            store(i - _NB).wait()
        obuf[sl] = xbuf[sl] + pooled
        store(i).start()
        if i + _NB < nch:
            load(i + _NB).start()

    for i in range(max(0, nch - _NB), nch):
        store(i).wait()


def kernel(x, table):
    B, S, d = x.shape
    V = table.shape[0]
    N = B * S
    xf = x.reshape(N, d)
    body = functools.partial(_body, S=S, N=N)
    out = pl.pallas_call(
        body,
        in_specs=[
            pl.BlockSpec(memory_space=pl.ANY),
            pl.BlockSpec((V, d), lambda: (0, 0)),
        ],
        out_specs=pl.BlockSpec(memory_space=pl.ANY),
        out_shape=jax.ShapeDtypeStruct((N, d), x.dtype),
        scratch_shapes=[
            pltpu.VMEM((_NB, _CH, d), jnp.float32),
            pltpu.VMEM((_NB, _CH, d), jnp.float32),
            pltpu.SemaphoreType.DMA((_NB,)),
            pltpu.SemaphoreType.DMA((_NB,)),
        ],
    )(xf, table)
    return out.reshape(B, S, d)


# speedup vs baseline: 5.2050x; 1.0308x over previous
import functools

import jax
import jax.numpy as jnp
from jax.experimental import pallas as pl
from jax.experimental.pallas import tpu as pltpu

_D = 768
_MAX_REL = 125
_VOCAB = 2 * _MAX_REL + 1
_CH = 2048
_NB = 2


def _body(x_hbm, table_ref, out_hbm, xbuf, obuf, insems, outsems, *, S, N):
    nch = N // _CH
    tbl = table_ref[...]

    def load(i):
        sl = i % _NB
        return pltpu.make_async_copy(
            x_hbm.at[pl.ds(i * _CH, _CH), :], xbuf.at[sl], insems.at[sl])

    def store(i):
        sl = i % _NB
        return pltpu.make_async_copy(
            obuf.at[sl], out_hbm.at[pl.ds(i * _CH, _CH), :], outsems.at[sl])

    for i in range(min(_NB, nch)):
        load(i).start()

    for i in range(nch):
        sl = i % _NB
        load(i).wait()
        rows = i * _CH + jax.lax.broadcasted_iota(jnp.int32, (_CH, _VOCAB), 0)
        jj = jax.lax.bitwise_and(rows, S - 1)
        kk = jax.lax.broadcasted_iota(jnp.int32, (_CH, _VOCAB), 1)
        interior = ((kk >= 1) & (kk <= _VOCAB - 2)
                    & (jj >= kk - _MAX_REL) & (jj <= kk - _MAX_REL + S - 1))
        cnt = jnp.where(kk == 0, jnp.maximum(0, (S - _MAX_REL) - jj), 0)
        cnt = cnt + jnp.where(kk == _VOCAB - 1,
                              jnp.maximum(0, jj - (_MAX_REL - 1)), 0)
        cnt = cnt + interior.astype(jnp.int32)
        c = cnt.astype(jnp.float32) * (1.0 / S)
        pooled = jax.lax.dot_general(
            c, tbl,
            dimension_numbers=(((1,), (0,)), ((), ())),
            preferred_element_type=jnp.float32,
        )
        if i >= _NB:
            store(i - _NB).wait()
        obuf[sl] = xbuf[sl] + pooled
        store(i).start()
        if i + _NB < nch:
            load(i + _NB).start()

    for i in range(max(0, nch - _NB), nch):
        store(i).wait()


def kernel(x, table):
    B, S, d = x.shape
    V = table.shape[0]
    N = B * S
    xf = x.reshape(N, d)
    body = functools.partial(_body, S=S, N=N)
    out = pl.pallas_call(
        body,
        in_specs=[
            pl.BlockSpec(memory_space=pl.ANY),
            pl.BlockSpec((V, d), lambda: (0, 0)),
        ],
        out_specs=pl.BlockSpec(memory_space=pl.ANY),
        out_shape=jax.ShapeDtypeStruct((N, d), x.dtype),
        scratch_shapes=[
            pltpu.VMEM((_NB, _CH, d), jnp.float32),
            pltpu.VMEM((_NB, _CH, d), jnp.float32),
            pltpu.SemaphoreType.DMA((_NB,)),
            pltpu.SemaphoreType.DMA((_NB,)),
        ],
    )(xf, table)
    return out.reshape(B, S, d)
